# baseline (speedup 1.0000x reference)
import jax
import jax.numpy as jnp
from jax import lax
from jax.experimental import pallas as pl
from jax.experimental.pallas import tpu as pltpu
from jax.experimental.pallas import tpu_sc as plsc

N = 50000
E = 800000
MAX_DEG = 512
HID = 128
OUT = 128
NLAYERS = 5

GPT = 400
EP = 16 * GPT * 128
GROUPS = EP // 128
DUMMY_DST = N

SP_ROWS = 50176
ZPT = SP_ROWS // 16
NCHUNK = 391

BN = 2000
NB = N // BN

_mesh = plsc.VectorSubcoreMesh(core_axis_name="c", subcore_axis_name="s")
_sc_params = pltpu.CompilerParams(use_tc_tiling_on_sc=False)


def _ds8(off, n):
    return pl.ds(pl.multiple_of(off, 8), n)


def _zero_agg_slice(zbuf, agg, s):
    zb = s * ZPT

    def zcp(k, carry):
        pltpu.sync_copy(zbuf, agg.at[_ds8(zb + k * 128, 128)])
        return carry

    lax.fori_loop(0, 24, zcp, 0)
    pltpu.sync_copy(zbuf.at[pl.ds(0, 64)], agg.at[_ds8(zb + 3072, 64)])


def _copy_out_slice(agg, out, s, out_base=0):
    def ocp(k, carry):
        q = s + 16 * k

        @pl.when(q < NCHUNK - 1)
        def _():
            pltpu.sync_copy(agg.at[_ds8(q * 128, 128)],
                            out.at[_ds8(out_base + q * 128, 128)])

        @pl.when(q == NCHUNK - 1)
        def _():
            pltpu.sync_copy(agg.at[_ds8(49920, 80)],
                            out.at[_ds8(out_base + 49920, 80)])

        return carry

    lax.fori_loop(0, 25, ocp, 0)


HGRP = GPT // 2


def _edge_pass(h, src2d, dst2d, sidx, didx, rows, agg, gsem, ssem, s):
    for hf in range(2):
        base = s * GPT + hf * HGRP
        pltpu.sync_copy(src2d.at[_ds8(base, HGRP)], sidx)
        pltpu.sync_copy(dst2d.at[_ds8(base, HGRP)], didx)
        for u in (0, 1):
            pltpu.async_copy(h.at[sidx.at[u]], rows.at[u], gsem[u])

        def quad(q, carry):
            for u in range(4):
                g = 4 * q + u
                r2 = (u + 2) % 4
                @pl.when(g + 2 < HGRP)
                def _(g=g, u=u, r2=r2):
                    @pl.when(g >= 2)
                    def _():
                        pltpu.make_async_copy(rows.at[r2],
                                              agg.at[didx.at[g]],
                                              ssem[r2]).wait()
                    pltpu.async_copy(h.at[sidx.at[g + 2]], rows.at[r2],
                                     gsem[r2])
                pltpu.make_async_copy(h.at[sidx.at[g]], rows.at[u],
                                      gsem[u]).wait()
                pltpu.async_copy(rows.at[u], agg.at[didx.at[g]],
                                 ssem[u], add=True)
            return carry

        lax.fori_loop(0, HGRP // 4, quad, 0)
        for u in range(4):
            pltpu.make_async_copy(rows.at[u],
                                  agg.at[didx.at[HGRP - 4 + u]],
                                  ssem[u]).wait()


def _make_segsum(nstrips):
    nsrc = (nstrips + 1) // 2
    assign = ([k for k in range(nstrips) if k % 2 == 0],
              [k for k in range(nstrips) if k % 2 == 1])
    npass = len(assign[0])

    def body(*refs):
        hv = refs[0:nsrc]
        srcA2d, srcB2d, dst2d, zeros_h = refs[nsrc:nsrc + 4]
        outs = refs[nsrc + 4:nsrc + 4 + nstrips]
        (sidx, didx, rows, zbuf, agg,
         g0, g1, g2, g3, s0, s1_, s2_, s3) = refs[nsrc + 4 + nstrips:]
        gsem = (g0, g1, g2, g3)
        ssem = (s0, s1_, s2_, s3)
        c = lax.axis_index("c")
        s = lax.axis_index("s")
        pltpu.sync_copy(zeros_h, zbuf)

        for p in range(npass):
            for cc in range(2):
                if p < len(assign[cc]):
                    @pl.when(c == cc)
                    def _(s=s):
                        _zero_agg_slice(zbuf, agg, s)
            plsc.subcore_barrier()
            for cc in range(2):
                if p < len(assign[cc]):
                    sid = assign[cc][p]
                    src2d = srcA2d if sid % 2 == 0 else srcB2d

                    @pl.when(c == cc)
                    def _(sid=sid, src2d=src2d, s=s):
                        _edge_pass(hv[sid // 2], src2d, dst2d, sidx, didx,
                                   rows, agg, gsem, ssem, s)
            plsc.subcore_barrier()
            for cc in range(2):
                if p < len(assign[cc]):
                    sid = assign[cc][p]

                    @pl.when(c == cc)
                    def _(sid=sid, s=s):
                        _copy_out_slice(agg, outs[sid], s)
            plsc.subcore_barrier()

    return pl.kernel(
        body,
        out_type=[jax.ShapeDtypeStruct((N, 16), jnp.float32)
                  for _ in range(nstrips)],
        mesh=_mesh,
        compiler_params=_sc_params,
        scratch_types=[
            pltpu.VMEM((HGRP, 128), jnp.int32),
            pltpu.VMEM((HGRP, 128), jnp.int32),
            pltpu.VMEM((4, 128, 16), jnp.float32),
            pltpu.VMEM((128, 16), jnp.float32),
            pltpu.VMEM_SHARED((SP_ROWS, 16), jnp.float32),
        ] + [pltpu.SemaphoreType.DMA] * 8,
    )


_segsum5 = _make_segsum(5)
_segsum8 = _make_segsum(8)


def _deg_body(dst2d, ones_h, zeros_h, out, dblk, ones_v, zbuf, agg,
              ssem0, ssem1):
    ssem = (ssem0, ssem1)
    c = lax.axis_index("c")
    s = lax.axis_index("s")
    pltpu.sync_copy(zeros_h, zbuf)
    pltpu.sync_copy(ones_h, ones_v)
    _zero_agg_slice(zbuf, agg, s)
    plsc.subcore_barrier()
    gbase = c * (GROUPS // 2) + s * (GPT // 2)

    def drain(par):
        for g in range(8):
            pltpu.make_async_copy(ones_v, agg.at[dblk.at[par, g]],
                                  ssem[par]).wait()

    def step(bi, par, guard):
        if guard:
            @pl.when(bi >= 2)
            def _():
                drain(par)
        else:
            drain(par)
        pltpu.sync_copy(dst2d.at[_ds8(gbase + bi * 8, 8)], dblk.at[par])
        for g in range(8):
            pltpu.async_copy(ones_v, agg.at[dblk.at[par, g]],
                             ssem[par], add=True)

    def dbl(t, carry):
        step(2 * t, 0, True)
        step(2 * t + 1, 1, True)
        return carry

    lax.fori_loop(0, 12, dbl, 0)
    step(24, 0, False)
    drain(1)
    drain(0)
    plsc.subcore_barrier()
    _copy_out_slice(agg, out, s, out_base=c * N)


_deg_kernel = pl.kernel(
    _deg_body,
    out_type=jax.ShapeDtypeStruct((2 * N, 16), jnp.float32),
    mesh=_mesh,
    compiler_params=_sc_params,
    scratch_types=[
        pltpu.VMEM((2, 8, 128), jnp.int32),
        pltpu.VMEM((128, 16), jnp.float32),
        pltpu.VMEM((128, 16), jnp.float32),
        pltpu.VMEM_SHARED((SP_ROWS, 16), jnp.float32),
        pltpu.SemaphoreType.DMA,
        pltpu.SemaphoreType.DMA,
    ],
)


def _h0_body(nt_ref, d0_ref, d1_ref, pos_ref, ast_t, deg_t,
             s1_ref, s2_ref, pooled_ref):
    i = pl.program_id(0)
    nt = nt_ref[0, 0, :]
    deg = (d0_ref[:, 0] + d1_ref[:, 0]).astype(jnp.int32)
    degc = jnp.clip(deg, 0, MAX_DEG)
    oh_a = (lax.broadcasted_iota(jnp.int32, (BN, 128), 1)
            == nt[:, None]).astype(jnp.float32)
    ast_e = lax.dot_general(oh_a, ast_t[...], (((1,), (0,)), ((), ())),
                            preferred_element_type=jnp.float32)
    oh_d = (lax.broadcasted_iota(jnp.int32, (BN, 520), 1)
            == degc[:, None]).astype(jnp.float32)
    deg_e = lax.dot_general(oh_d, deg_t[...], (((1,), (0,)), ((), ())),
                            preferred_element_type=jnp.float32)
    s1 = jnp.concatenate([deg_e, ast_e[:, :16]], axis=1)
    s2 = jnp.concatenate([ast_e[:, 16:], jnp.zeros((BN, 16), jnp.float32)],
                         axis=1)
    s1_ref[...] = s1
    s2_ref[...] = s2
    colsum = jnp.sum(jnp.concatenate(
        [pos_ref[...], s1, s2, jnp.zeros((BN, 32), jnp.float32)], axis=1),
        axis=0)
    row = jnp.where(lax.broadcasted_iota(jnp.int32, (8, OUT), 0) == 0,
                    colsum[None, :], 0.0)

    @pl.when(i == 0)
    def _():
        pooled_ref[...] = row

    @pl.when(i != 0)
    def _():
        pooled_ref[...] += row


def _mlp_body_factory(kh, ka):
    def body(*refs):
        i = pl.program_id(0)
        h_refs = refs[0:kh]
        a_refs = refs[kh:kh + ka]
        w1_ref, b1_ref, w2_ref, b2_ref = refs[kh + ka:kh + ka + 4]
        z_refs = refs[kh + ka + 4:kh + ka + 8]
        pooled_ref = refs[kh + ka + 8]
        hparts = [h_refs[j][...] for j in range(kh)]
        if kh < 4:
            hparts.append(jnp.zeros((BN, 32 * (4 - kh)), jnp.float32))
        aparts = [a_refs[j][...] for j in range(ka)]
        if ka < 8:
            aparts.append(jnp.zeros((BN, 16 * (8 - ka)), jnp.float32))
        x = (jnp.concatenate(hparts, axis=1)
             + jnp.concatenate(aparts, axis=1))
        y = lax.dot_general(x, w1_ref[...], (((1,), (0,)), ((), ())),
                            preferred_element_type=jnp.float32)
        y = jax.nn.relu(y + b1_ref[...])
        z = lax.dot_general(y, w2_ref[...], (((1,), (0,)), ((), ())),
                            preferred_element_type=jnp.float32)
        z = jax.nn.relu(z + b2_ref[...])
        for sjj in range(4):
            z_refs[sjj][...] = z[:, sjj * 32:(sjj + 1) * 32]
        colsum = jnp.sum(z, axis=0)
        row = jnp.where(lax.broadcasted_iota(jnp.int32, (8, OUT), 0) == 0,
                        colsum[None, :], 0.0)

        @pl.when(i == 0)
        def _():
            pooled_ref[...] = row

        @pl.when(i != 0)
        def _():
            pooled_ref[...] += row

    return body


def _make_mlp(kh, ka):
    bspec32 = pl.BlockSpec((BN, 32), lambda i: (i, 0))
    bspec16 = pl.BlockSpec((BN, 16), lambda i: (i, 0))
    wspec = pl.BlockSpec((HID, HID), lambda i: (0, 0))
    bvecspec = pl.BlockSpec((1, HID), lambda i: (0, 0))
    return pl.pallas_call(
        _mlp_body_factory(kh, ka),
        grid=(NB,),
        in_specs=[bspec32] * kh + [bspec16] * ka
        + [wspec, bvecspec, wspec, bvecspec],
        out_specs=[bspec32] * 4 + [pl.BlockSpec((8, OUT), lambda i: (0, 0))],
        out_shape=[jax.ShapeDtypeStruct((N, 32), jnp.float32)] * 4
        + [jax.ShapeDtypeStruct((8, OUT), jnp.float32)],
    )


_mlp1 = _make_mlp(3, 5)
_mlp = _make_mlp(4, 8)

_h0_call = pl.pallas_call(
    _h0_body,
    grid=(NB,),
    in_specs=[
        pl.BlockSpec((1, 1, BN), lambda i: (i, 0, 0)),
        pl.BlockSpec((BN, 16), lambda i: (i, 0)),
        pl.BlockSpec((BN, 16), lambda i: (i + NB, 0)),
        pl.BlockSpec((BN, 32), lambda i: (i, 0)),
        pl.BlockSpec((128, 32), lambda i: (0, 0)),
        pl.BlockSpec((520, 16), lambda i: (0, 0)),
    ],
    out_specs=[
        pl.BlockSpec((BN, 32), lambda i: (i, 0)),
        pl.BlockSpec((BN, 32), lambda i: (i, 0)),
        pl.BlockSpec((8, OUT), lambda i: (0, 0)),
    ],
    out_shape=[
        jax.ShapeDtypeStruct((N, 32), jnp.float32),
        jax.ShapeDtypeStruct((N, 32), jnp.float32),
        jax.ShapeDtypeStruct((8, OUT), jnp.float32),
    ],
)


def _readout_body(pooled_ref, w_ref, b_ref, out_ref):
    acc = jnp.zeros((8, OUT), jnp.float32)
    for i in range(NLAYERS + 1):
        acc = acc + lax.dot_general(
            pooled_ref[i], w_ref[i], (((1,), (0,)), ((), ())),
            preferred_element_type=jnp.float32)
    bsum = jnp.sum(b_ref[...], axis=0)
    out_ref[...] = acc + jnp.broadcast_to(bsum[None, :], (8, OUT))


_readout = pl.pallas_call(
    _readout_body,
    out_shape=jax.ShapeDtypeStruct((8, OUT), jnp.float32),
)


def kernel(node_type, pos_undirected, edge_index, ast_table, deg_table,
           gin_w1, gin_b1, gin_w2, gin_b2, pred_w, pred_b):
    src = edge_index[0]
    dst = edge_index[1]
    npad = EP - E
    srcA2d = jnp.concatenate(
        [2 * src, jnp.zeros((npad,), jnp.int32)]).reshape(GROUPS, 128)
    srcB2d = jnp.concatenate(
        [2 * src + 1, jnp.ones((npad,), jnp.int32)]).reshape(GROUPS, 128)
    dst2d = jnp.concatenate(
        [dst, jnp.full((npad,), DUMMY_DST, jnp.int32)]).reshape(GROUPS, 128)

    zeros16 = jnp.zeros((128, 16), jnp.float32)
    ones16 = jnp.ones((128, 16), jnp.float32)

    deg2 = _deg_kernel(dst2d, ones16, zeros16)

    nt3 = node_type.reshape(NB, 1, BN)
    ast_pad = jnp.zeros((128, 32), jnp.float32).at[:101].set(ast_table)
    degtab_pad = jnp.zeros((520, 16), jnp.float32).at[:513].set(deg_table)
    s1, s2, pooled0 = _h0_call(nt3, deg2, deg2, pos_undirected,
                               ast_pad, degtab_pad)

    pooleds = [pooled0]
    h0v = [pos_undirected.reshape(2 * N, 16), s1.reshape(2 * N, 16),
           s2.reshape(2 * N, 16)]
    a = _segsum5(h0v[0], h0v[1], h0v[2], srcA2d, srcB2d, dst2d, zeros16)
    w1p = jnp.zeros((HID, HID), jnp.float32).at[:80].set(gin_w1[0])
    z0, z1, z2, z3, pooled = _mlp1(
        pos_undirected, s1, s2, a[0], a[1], a[2], a[3], a[4],
        w1p, gin_b1[0].reshape(1, HID), gin_w2[0], gin_b2[0].reshape(1, HID))
    pooleds.append(pooled)
    hs = [z0, z1, z2, z3]
    for li in range(1, NLAYERS):
        hv = [h.reshape(2 * N, 16) for h in hs]
        a = _segsum8(hv[0], hv[1], hv[2], hv[3],
                     srcA2d, srcB2d, dst2d, zeros16)
        z0, z1, z2, z3, pooled = _mlp(
            hs[0], hs[1], hs[2], hs[3],
            a[0], a[1], a[2], a[3], a[4], a[5], a[6], a[7],
            gin_w1[li], gin_b1[li].reshape(1, HID),
            gin_w2[li], gin_b2[li].reshape(1, HID))
        pooleds.append(pooled)
        hs = [z0, z1, z2, z3]

    pooled_arr = jnp.stack(pooleds)
    w_pad = [jnp.zeros((OUT, OUT), jnp.float32).at[:80].set(pred_w[0])]
    w_pad += [pred_w[i] for i in range(1, NLAYERS + 1)]
    w_arr = jnp.stack(w_pad)
    b_arr = jnp.zeros((8, OUT), jnp.float32).at[:NLAYERS + 1].set(
        jnp.stack(pred_b))
    out = _readout(pooled_arr, w_arr, b_arr)
    return out[:1]


# speedup vs baseline: 1.0273x; 1.0273x over previous
import jax
import jax.numpy as jnp
from jax import lax
from jax.experimental import pallas as pl
from jax.experimental.pallas import tpu as pltpu
from jax.experimental.pallas import tpu_sc as plsc

N = 50000
E = 800000
MAX_DEG = 512
HID = 128
OUT = 128
NLAYERS = 5

GPT = 400
EP = 16 * GPT * 128
GROUPS = EP // 128
DUMMY_DST = N

SP_ROWS = 50176
ZPT = SP_ROWS // 16
NCHUNK = 391

BN = 2000
NB = N // BN

_mesh = plsc.VectorSubcoreMesh(core_axis_name="c", subcore_axis_name="s")
_sc_params = pltpu.CompilerParams(use_tc_tiling_on_sc=False)


def _ds8(off, n):
    return pl.ds(pl.multiple_of(off, 8), n)


def _zero_agg_slice(zbuf, agg, s):
    zb = s * ZPT

    def zcp(k, carry):
        pltpu.sync_copy(zbuf, agg.at[_ds8(zb + k * 128, 128)])
        return carry

    lax.fori_loop(0, 24, zcp, 0)
    pltpu.sync_copy(zbuf.at[pl.ds(0, 64)], agg.at[_ds8(zb + 3072, 64)])


def _copy_out_slice(agg, out, s, out_base=0):
    def ocp(k, carry):
        q = s + 16 * k

        @pl.when(q < NCHUNK - 1)
        def _():
            pltpu.sync_copy(agg.at[_ds8(q * 128, 128)],
                            out.at[_ds8(out_base + q * 128, 128)])

        @pl.when(q == NCHUNK - 1)
        def _():
            pltpu.sync_copy(agg.at[_ds8(49920, 80)],
                            out.at[_ds8(out_base + 49920, 80)])

        return carry

    lax.fori_loop(0, 25, ocp, 0)


def _edge_pass(h, src2d, dst2d, sblk, dblk, rows, agg, gsem, ssem, s):
    gbase = s * GPT
    nblk = GPT // 8

    def fire_gathers(bi, par):
        pltpu.sync_copy(src2d.at[_ds8(gbase + bi * 8, 8)], sblk.at[par])
        pltpu.sync_copy(dst2d.at[_ds8(gbase + bi * 8, 8)], dblk.at[par])
        for g in range(8):
            pltpu.async_copy(h.at[sblk.at[par, g]], rows.at[par, g],
                             gsem[par])

    def drain_scatters(par):
        for g in range(8):
            pltpu.make_async_copy(rows.at[par, g], agg.at[dblk.at[par, g]],
                                  ssem[par]).wait()

    def step(bi, par):
        @pl.when(bi >= 1)
        def _():
            drain_scatters(1 - par)

        @pl.when(bi < nblk - 1)
        def _():
            fire_gathers(bi + 1, 1 - par)

        for g in range(8):
            pltpu.make_async_copy(h.at[sblk.at[par, g]], rows.at[par, g],
                                  gsem[par]).wait()
        for g in range(8):
            pltpu.async_copy(rows.at[par, g], agg.at[dblk.at[par, g]],
                             ssem[par], add=True)

    fire_gathers(0, 0)

    def dbl(t, carry):
        step(2 * t, 0)
        step(2 * t + 1, 1)
        return carry

    lax.fori_loop(0, nblk // 2, dbl, 0)
    drain_scatters(1)


def _make_segsum(nstrips):
    nsrc = (nstrips + 1) // 2
    assign = ([k for k in range(nstrips) if k % 2 == 0],
              [k for k in range(nstrips) if k % 2 == 1])
    npass = len(assign[0])

    def body(*refs):
        hv = refs[0:nsrc]
        srcA2d, srcB2d, dst2d, zeros_h = refs[nsrc:nsrc + 4]
        outs = refs[nsrc + 4:nsrc + 4 + nstrips]
        scr = refs[nsrc + 4 + nstrips:]
        sblk, dblk, rows, zbuf, agg = scr[:5]
        gsem = tuple(scr[5:7])
        ssem = tuple(scr[7:9])
        c = lax.axis_index("c")
        s = lax.axis_index("s")
        pltpu.sync_copy(zeros_h, zbuf)

        for p in range(npass):
            for cc in range(2):
                if p < len(assign[cc]):
                    @pl.when(c == cc)
                    def _(s=s):
                        _zero_agg_slice(zbuf, agg, s)
            plsc.subcore_barrier()
            for cc in range(2):
                if p < len(assign[cc]):
                    sid = assign[cc][p]
                    src2d = srcA2d if sid % 2 == 0 else srcB2d

                    @pl.when(c == cc)
                    def _(sid=sid, src2d=src2d, s=s):
                        _edge_pass(hv[sid // 2], src2d, dst2d, sblk, dblk,
                                   rows, agg, gsem, ssem, s)
            plsc.subcore_barrier()
            for cc in range(2):
                if p < len(assign[cc]):
                    sid = assign[cc][p]

                    @pl.when(c == cc)
                    def _(sid=sid, s=s):
                        _copy_out_slice(agg, outs[sid], s)
            plsc.subcore_barrier()

    return pl.kernel(
        body,
        out_type=[jax.ShapeDtypeStruct((N, 16), jnp.float32)
                  for _ in range(nstrips)],
        mesh=_mesh,
        compiler_params=_sc_params,
        scratch_types=[
            pltpu.VMEM((2, 8, 128), jnp.int32),
            pltpu.VMEM((2, 8, 128), jnp.int32),
            pltpu.VMEM((2, 8, 128, 16), jnp.float32),
            pltpu.VMEM((128, 16), jnp.float32),
            pltpu.VMEM_SHARED((SP_ROWS, 16), jnp.float32),
        ] + [pltpu.SemaphoreType.DMA] * 4,
    )


_segsum5 = _make_segsum(5)
_segsum8 = _make_segsum(8)


def _deg_body(dst2d, ones_h, zeros_h, out, dblk, ones_v, zbuf, agg,
              ssem0, ssem1):
    ssem = (ssem0, ssem1)
    c = lax.axis_index("c")
    s = lax.axis_index("s")
    pltpu.sync_copy(zeros_h, zbuf)
    pltpu.sync_copy(ones_h, ones_v)
    _zero_agg_slice(zbuf, agg, s)
    plsc.subcore_barrier()
    gbase = c * (GROUPS // 2) + s * (GPT // 2)

    def drain(par):
        for g in range(8):
            pltpu.make_async_copy(ones_v, agg.at[dblk.at[par, g]],
                                  ssem[par]).wait()

    def step(bi, par, guard):
        if guard:
            @pl.when(bi >= 2)
            def _():
                drain(par)
        else:
            drain(par)
        pltpu.sync_copy(dst2d.at[_ds8(gbase + bi * 8, 8)], dblk.at[par])
        for g in range(8):
            pltpu.async_copy(ones_v, agg.at[dblk.at[par, g]],
                             ssem[par], add=True)

    def dbl(t, carry):
        step(2 * t, 0, True)
        step(2 * t + 1, 1, True)
        return carry

    lax.fori_loop(0, 12, dbl, 0)
    step(24, 0, False)
    drain(1)
    drain(0)
    plsc.subcore_barrier()
    _copy_out_slice(agg, out, s, out_base=c * N)


_deg_kernel = pl.kernel(
    _deg_body,
    out_type=jax.ShapeDtypeStruct((2 * N, 16), jnp.float32),
    mesh=_mesh,
    compiler_params=_sc_params,
    scratch_types=[
        pltpu.VMEM((2, 8, 128), jnp.int32),
        pltpu.VMEM((128, 16), jnp.float32),
        pltpu.VMEM((128, 16), jnp.float32),
        pltpu.VMEM_SHARED((SP_ROWS, 16), jnp.float32),
        pltpu.SemaphoreType.DMA,
        pltpu.SemaphoreType.DMA,
    ],
)


def _h0_body(nt_ref, d0_ref, d1_ref, pos_ref, ast_t, deg_t,
             s1_ref, s2_ref, pooled_ref):
    i = pl.program_id(0)
    nt = nt_ref[0, 0, :]
    deg = (d0_ref[:, 0] + d1_ref[:, 0]).astype(jnp.int32)
    degc = jnp.clip(deg, 0, MAX_DEG)
    oh_a = (lax.broadcasted_iota(jnp.int32, (BN, 128), 1)
            == nt[:, None]).astype(jnp.float32)
    ast_e = lax.dot_general(oh_a, ast_t[...], (((1,), (0,)), ((), ())),
                            preferred_element_type=jnp.float32)
    oh_d = (lax.broadcasted_iota(jnp.int32, (BN, 520), 1)
            == degc[:, None]).astype(jnp.float32)
    deg_e = lax.dot_general(oh_d, deg_t[...], (((1,), (0,)), ((), ())),
                            preferred_element_type=jnp.float32)
    s1 = jnp.concatenate([deg_e, ast_e[:, :16]], axis=1)
    s2 = jnp.concatenate([ast_e[:, 16:], jnp.zeros((BN, 16), jnp.float32)],
                         axis=1)
    s1_ref[...] = s1
    s2_ref[...] = s2
    colsum = jnp.sum(jnp.concatenate(
        [pos_ref[...], s1, s2, jnp.zeros((BN, 32), jnp.float32)], axis=1),
        axis=0)
    row = jnp.where(lax.broadcasted_iota(jnp.int32, (8, OUT), 0) == 0,
                    colsum[None, :], 0.0)

    @pl.when(i == 0)
    def _():
        pooled_ref[...] = row

    @pl.when(i != 0)
    def _():
        pooled_ref[...] += row


def _mlp_body_factory(kh, ka):
    def body(*refs):
        i = pl.program_id(0)
        h_refs = refs[0:kh]
        a_refs = refs[kh:kh + ka]
        w1_ref, b1_ref, w2_ref, b2_ref = refs[kh + ka:kh + ka + 4]
        z_refs = refs[kh + ka + 4:kh + ka + 8]
        pooled_ref = refs[kh + ka + 8]
        hparts = [h_refs[j][...] for j in range(kh)]
        if kh < 4:
            hparts.append(jnp.zeros((BN, 32 * (4 - kh)), jnp.float32))
        aparts = [a_refs[j][...] for j in range(ka)]
        if ka < 8:
            aparts.append(jnp.zeros((BN, 16 * (8 - ka)), jnp.float32))
        x = (jnp.concatenate(hparts, axis=1)
             + jnp.concatenate(aparts, axis=1))
        y = lax.dot_general(x, w1_ref[...], (((1,), (0,)), ((), ())),
                            preferred_element_type=jnp.float32)
        y = jax.nn.relu(y + b1_ref[...])
        z = lax.dot_general(y, w2_ref[...], (((1,), (0,)), ((), ())),
                            preferred_element_type=jnp.float32)
        z = jax.nn.relu(z + b2_ref[...])
        for sjj in range(4):
            z_refs[sjj][...] = z[:, sjj * 32:(sjj + 1) * 32]
        colsum = jnp.sum(z, axis=0)
        row = jnp.where(lax.broadcasted_iota(jnp.int32, (8, OUT), 0) == 0,
                        colsum[None, :], 0.0)

        @pl.when(i == 0)
        def _():
            pooled_ref[...] = row

        @pl.when(i != 0)
        def _():
            pooled_ref[...] += row

    return body


def _make_mlp(kh, ka):
    bspec32 = pl.BlockSpec((BN, 32), lambda i: (i, 0))
    bspec16 = pl.BlockSpec((BN, 16), lambda i: (i, 0))
    wspec = pl.BlockSpec((HID, HID), lambda i: (0, 0))
    bvecspec = pl.BlockSpec((1, HID), lambda i: (0, 0))
    return pl.pallas_call(
        _mlp_body_factory(kh, ka),
        grid=(NB,),
        in_specs=[bspec32] * kh + [bspec16] * ka
        + [wspec, bvecspec, wspec, bvecspec],
        out_specs=[bspec32] * 4 + [pl.BlockSpec((8, OUT), lambda i: (0, 0))],
        out_shape=[jax.ShapeDtypeStruct((N, 32), jnp.float32)] * 4
        + [jax.ShapeDtypeStruct((8, OUT), jnp.float32)],
    )


_mlp1 = _make_mlp(3, 5)
_mlp = _make_mlp(4, 8)

_h0_call = pl.pallas_call(
    _h0_body,
    grid=(NB,),
    in_specs=[
        pl.BlockSpec((1, 1, BN), lambda i: (i, 0, 0)),
        pl.BlockSpec((BN, 16), lambda i: (i, 0)),
        pl.BlockSpec((BN, 16), lambda i: (i + NB, 0)),
        pl.BlockSpec((BN, 32), lambda i: (i, 0)),
        pl.BlockSpec((128, 32), lambda i: (0, 0)),
        pl.BlockSpec((520, 16), lambda i: (0, 0)),
    ],
    out_specs=[
        pl.BlockSpec((BN, 32), lambda i: (i, 0)),
        pl.BlockSpec((BN, 32), lambda i: (i, 0)),
        pl.BlockSpec((8, OUT), lambda i: (0, 0)),
    ],
    out_shape=[
        jax.ShapeDtypeStruct((N, 32), jnp.float32),
        jax.ShapeDtypeStruct((N, 32), jnp.float32),
        jax.ShapeDtypeStruct((8, OUT), jnp.float32),
    ],
)


def _readout_body(pooled_ref, w_ref, b_ref, out_ref):
    acc = jnp.zeros((8, OUT), jnp.float32)
    for i in range(NLAYERS + 1):
        acc = acc + lax.dot_general(
            pooled_ref[i], w_ref[i], (((1,), (0,)), ((), ())),
            preferred_element_type=jnp.float32)
    bsum = jnp.sum(b_ref[...], axis=0)
    out_ref[...] = acc + jnp.broadcast_to(bsum[None, :], (8, OUT))


_readout = pl.pallas_call(
    _readout_body,
    out_shape=jax.ShapeDtypeStruct((8, OUT), jnp.float32),
)


def kernel(node_type, pos_undirected, edge_index, ast_table, deg_table,
           gin_w1, gin_b1, gin_w2, gin_b2, pred_w, pred_b):
    src = edge_index[0]
    dst = edge_index[1]
    npad = EP - E
    srcA2d = jnp.concatenate(
        [2 * src, jnp.zeros((npad,), jnp.int32)]).reshape(GROUPS, 128)
    srcB2d = jnp.concatenate(
        [2 * src + 1, jnp.ones((npad,), jnp.int32)]).reshape(GROUPS, 128)
    dst2d = jnp.concatenate(
        [dst, jnp.full((npad,), DUMMY_DST, jnp.int32)]).reshape(GROUPS, 128)

    zeros16 = jnp.zeros((128, 16), jnp.float32)
    ones16 = jnp.ones((128, 16), jnp.float32)

    deg2 = _deg_kernel(dst2d, ones16, zeros16)

    nt3 = node_type.reshape(NB, 1, BN)
    ast_pad = jnp.zeros((128, 32), jnp.float32).at[:101].set(ast_table)
    degtab_pad = jnp.zeros((520, 16), jnp.float32).at[:513].set(deg_table)
    s1, s2, pooled0 = _h0_call(nt3, deg2, deg2, pos_undirected,
                               ast_pad, degtab_pad)

    pooleds = [pooled0]
    h0v = [pos_undirected.reshape(2 * N, 16), s1.reshape(2 * N, 16),
           s2.reshape(2 * N, 16)]
    a = _segsum5(h0v[0], h0v[1], h0v[2], srcA2d, srcB2d, dst2d, zeros16)
    w1p = jnp.zeros((HID, HID), jnp.float32).at[:80].set(gin_w1[0])
    z0, z1, z2, z3, pooled = _mlp1(
        pos_undirected, s1, s2, a[0], a[1], a[2], a[3], a[4],
        w1p, gin_b1[0].reshape(1, HID), gin_w2[0], gin_b2[0].reshape(1, HID))
    pooleds.append(pooled)
    hs = [z0, z1, z2, z3]
    for li in range(1, NLAYERS):
        hv = [h.reshape(2 * N, 16) for h in hs]
        a = _segsum8(hv[0], hv[1], hv[2], hv[3],
                     srcA2d, srcB2d, dst2d, zeros16)
        z0, z1, z2, z3, pooled = _mlp(
            hs[0], hs[1], hs[2], hs[3],
            a[0], a[1], a[2], a[3], a[4], a[5], a[6], a[7],
            gin_w1[li], gin_b1[li].reshape(1, HID),
            gin_w2[li], gin_b2[li].reshape(1, HID))
        pooleds.append(pooled)
        hs = [z0, z1, z2, z3]

    pooled_arr = jnp.stack(pooleds)
    w_pad = [jnp.zeros((OUT, OUT), jnp.float32).at[:80].set(pred_w[0])]
    w_pad += [pred_w[i] for i in range(1, NLAYERS + 1)]
    w_arr = jnp.stack(w_pad)
    b_arr = jnp.zeros((8, OUT), jnp.float32).at[:NLAYERS + 1].set(
        jnp.stack(pred_b))
    out = _readout(pooled_arr, w_arr, b_arr)
    return out[:1]


# speedup vs baseline: 1.0356x; 1.0081x over previous
import jax
import jax.numpy as jnp
from jax import lax
from jax.experimental import pallas as pl
from jax.experimental.pallas import tpu as pltpu
from jax.experimental.pallas import tpu_sc as plsc

N = 50000
E = 800000
MAX_DEG = 512
HID = 128
OUT = 128
NLAYERS = 5

GPT = 400
EP = 16 * GPT * 128
GROUPS = EP // 128
DUMMY_DST = N

SP_ROWS = 50176
ZPT = SP_ROWS // 16
NCHUNK = 391

BN = 2000
NB = N // BN

_mesh = plsc.VectorSubcoreMesh(core_axis_name="c", subcore_axis_name="s")
_sc_params = pltpu.CompilerParams(use_tc_tiling_on_sc=False)


def _ds8(off, n):
    return pl.ds(pl.multiple_of(off, 8), n)


def _zero_agg_slice(zbuf, agg, s):
    zb = s * ZPT

    def zcp(k, carry):
        pltpu.sync_copy(zbuf, agg.at[_ds8(zb + k * 128, 128)])
        return carry

    lax.fori_loop(0, 24, zcp, 0)
    pltpu.sync_copy(zbuf.at[pl.ds(0, 64)], agg.at[_ds8(zb + 3072, 64)])


def _copy_out_slice(agg, out, s, out_base=0):
    def ocp(k, carry):
        q = s + 16 * k

        @pl.when(q < NCHUNK - 1)
        def _():
            pltpu.sync_copy(agg.at[_ds8(q * 128, 128)],
                            out.at[_ds8(out_base + q * 128, 128)])

        @pl.when(q == NCHUNK - 1)
        def _():
            pltpu.sync_copy(agg.at[_ds8(49920, 80)],
                            out.at[_ds8(out_base + 49920, 80)])

        return carry

    lax.fori_loop(0, 25, ocp, 0)


def _edge_pass(h, src2d, dst2d, sblk, dblk, rows, agg, gsem, ssem, s):
    gbase = s * GPT
    nblk = GPT // 8

    def fire_gathers(bi, par):
        pltpu.sync_copy(src2d.at[_ds8(gbase + bi * 8, 8)], sblk.at[par])
        pltpu.sync_copy(dst2d.at[_ds8(gbase + bi * 8, 8)], dblk.at[par])
        for g in range(8):
            pltpu.async_copy(h.at[sblk.at[par, g]], rows.at[par, g],
                             gsem[par])

    def drain_scatters(par):
        for g in range(8):
            pltpu.make_async_copy(rows.at[par, g], agg.at[dblk.at[par, g]],
                                  ssem[par]).wait()

    def step(bi, par):
        @pl.when(bi >= 1)
        def _():
            drain_scatters(1 - par)

        @pl.when(bi < nblk - 1)
        def _():
            fire_gathers(bi + 1, 1 - par)

        for g in range(8):
            pltpu.make_async_copy(h.at[sblk.at[par, g]], rows.at[par, g],
                                  gsem[par]).wait()
            pltpu.async_copy(rows.at[par, g], agg.at[dblk.at[par, g]],
                             ssem[par], add=True)

    fire_gathers(0, 0)

    def dbl(t, carry):
        step(2 * t, 0)
        step(2 * t + 1, 1)
        return carry

    lax.fori_loop(0, nblk // 2, dbl, 0)
    drain_scatters(1)


def _make_segsum(nstrips):
    nsrc = (nstrips + 1) // 2
    assign = ([k for k in range(nstrips) if k % 2 == 0],
              [k for k in range(nstrips) if k % 2 == 1])
    npass = len(assign[0])

    def body(*refs):
        hv = refs[0:nsrc]
        srcA2d, srcB2d, dst2d, zeros_h = refs[nsrc:nsrc + 4]
        outs = refs[nsrc + 4:nsrc + 4 + nstrips]
        scr = refs[nsrc + 4 + nstrips:]
        sblk, dblk, rows, zbuf, agg = scr[:5]
        gsem = tuple(scr[5:7])
        ssem = tuple(scr[7:9])
        c = lax.axis_index("c")
        s = lax.axis_index("s")
        pltpu.sync_copy(zeros_h, zbuf)

        for p in range(npass):
            for cc in range(2):
                if p < len(assign[cc]):
                    @pl.when(c == cc)
                    def _(s=s):
                        _zero_agg_slice(zbuf, agg, s)
            plsc.subcore_barrier()
            for cc in range(2):
                if p < len(assign[cc]):
                    sid = assign[cc][p]
                    src2d = srcA2d if sid % 2 == 0 else srcB2d

                    @pl.when(c == cc)
                    def _(sid=sid, src2d=src2d, s=s):
                        _edge_pass(hv[sid // 2], src2d, dst2d, sblk, dblk,
                                   rows, agg, gsem, ssem, s)
            plsc.subcore_barrier()
            for cc in range(2):
                if p < len(assign[cc]):
                    sid = assign[cc][p]

                    @pl.when(c == cc)
                    def _(sid=sid, s=s):
                        _copy_out_slice(agg, outs[sid], s)
            plsc.subcore_barrier()

    return pl.kernel(
        body,
        out_type=[jax.ShapeDtypeStruct((N, 16), jnp.float32)
                  for _ in range(nstrips)],
        mesh=_mesh,
        compiler_params=_sc_params,
        scratch_types=[
            pltpu.VMEM((2, 8, 128), jnp.int32),
            pltpu.VMEM((2, 8, 128), jnp.int32),
            pltpu.VMEM((2, 8, 128, 16), jnp.float32),
            pltpu.VMEM((128, 16), jnp.float32),
            pltpu.VMEM_SHARED((SP_ROWS, 16), jnp.float32),
        ] + [pltpu.SemaphoreType.DMA] * 4,
    )


_segsum5 = _make_segsum(5)
_segsum8 = _make_segsum(8)


def _deg_body(dst2d, ones_h, zeros_h, out, dblk, ones_v, zbuf, agg,
              ssem0, ssem1):
    ssem = (ssem0, ssem1)
    c = lax.axis_index("c")
    s = lax.axis_index("s")
    pltpu.sync_copy(zeros_h, zbuf)
    pltpu.sync_copy(ones_h, ones_v)
    _zero_agg_slice(zbuf, agg, s)
    plsc.subcore_barrier()
    gbase = c * (GROUPS // 2) + s * (GPT // 2)

    def drain(par):
        for g in range(8):
            pltpu.make_async_copy(ones_v, agg.at[dblk.at[par, g]],
                                  ssem[par]).wait()

    def step(bi, par, guard):
        if guard:
            @pl.when(bi >= 2)
            def _():
                drain(par)
        else:
            drain(par)
        pltpu.sync_copy(dst2d.at[_ds8(gbase + bi * 8, 8)], dblk.at[par])
        for g in range(8):
            pltpu.async_copy(ones_v, agg.at[dblk.at[par, g]],
                             ssem[par], add=True)

    def dbl(t, carry):
        step(2 * t, 0, True)
        step(2 * t + 1, 1, True)
        return carry

    lax.fori_loop(0, 12, dbl, 0)
    step(24, 0, False)
    drain(1)
    drain(0)
    plsc.subcore_barrier()
    _copy_out_slice(agg, out, s, out_base=c * N)


_deg_kernel = pl.kernel(
    _deg_body,
    out_type=jax.ShapeDtypeStruct((2 * N, 16), jnp.float32),
    mesh=_mesh,
    compiler_params=_sc_params,
    scratch_types=[
        pltpu.VMEM((2, 8, 128), jnp.int32),
        pltpu.VMEM((128, 16), jnp.float32),
        pltpu.VMEM((128, 16), jnp.float32),
        pltpu.VMEM_SHARED((SP_ROWS, 16), jnp.float32),
        pltpu.SemaphoreType.DMA,
        pltpu.SemaphoreType.DMA,
    ],
)


def _h0_body(nt_ref, d0_ref, d1_ref, pos_ref, ast_t, deg_t,
             s1_ref, s2_ref, pooled_ref):
    i = pl.program_id(0)
    nt = nt_ref[0, 0, :]
    deg = (d0_ref[:, 0] + d1_ref[:, 0]).astype(jnp.int32)
    degc = jnp.clip(deg, 0, MAX_DEG)
    oh_a = (lax.broadcasted_iota(jnp.int32, (BN, 128), 1)
            == nt[:, None]).astype(jnp.float32)
    ast_e = lax.dot_general(oh_a, ast_t[...], (((1,), (0,)), ((), ())),
                            preferred_element_type=jnp.float32)
    oh_d = (lax.broadcasted_iota(jnp.int32, (BN, 520), 1)
            == degc[:, None]).astype(jnp.float32)
    deg_e = lax.dot_general(oh_d, deg_t[...], (((1,), (0,)), ((), ())),
                            preferred_element_type=jnp.float32)
    s1 = jnp.concatenate([deg_e, ast_e[:, :16]], axis=1)
    s2 = jnp.concatenate([ast_e[:, 16:], jnp.zeros((BN, 16), jnp.float32)],
                         axis=1)
    s1_ref[...] = s1
    s2_ref[...] = s2
    colsum = jnp.sum(jnp.concatenate(
        [pos_ref[...], s1, s2, jnp.zeros((BN, 32), jnp.float32)], axis=1),
        axis=0)
    row = jnp.where(lax.broadcasted_iota(jnp.int32, (8, OUT), 0) == 0,
                    colsum[None, :], 0.0)

    @pl.when(i == 0)
    def _():
        pooled_ref[...] = row

    @pl.when(i != 0)
    def _():
        pooled_ref[...] += row


def _mlp_body_factory(kh, ka):
    def body(*refs):
        i = pl.program_id(0)
        h_refs = refs[0:kh]
        a_refs = refs[kh:kh + ka]
        w1_ref, b1_ref, w2_ref, b2_ref = refs[kh + ka:kh + ka + 4]
        z_refs = refs[kh + ka + 4:kh + ka + 8]
        pooled_ref = refs[kh + ka + 8]
        hparts = [h_refs[j][...] for j in range(kh)]
        if kh < 4:
            hparts.append(jnp.zeros((BN, 32 * (4 - kh)), jnp.float32))
        aparts = [a_refs[j][...] for j in range(ka)]
        if ka < 8:
            aparts.append(jnp.zeros((BN, 16 * (8 - ka)), jnp.float32))
        x = (jnp.concatenate(hparts, axis=1)
             + jnp.concatenate(aparts, axis=1))
        y = lax.dot_general(x, w1_ref[...], (((1,), (0,)), ((), ())),
                            preferred_element_type=jnp.float32)
        y = jax.nn.relu(y + b1_ref[...])
        z = lax.dot_general(y, w2_ref[...], (((1,), (0,)), ((), ())),
                            preferred_element_type=jnp.float32)
        z = jax.nn.relu(z + b2_ref[...])
        for sjj in range(4):
            z_refs[sjj][...] = z[:, sjj * 32:(sjj + 1) * 32]
        colsum = jnp.sum(z, axis=0)
        row = jnp.where(lax.broadcasted_iota(jnp.int32, (8, OUT), 0) == 0,
                        colsum[None, :], 0.0)

        @pl.when(i == 0)
        def _():
            pooled_ref[...] = row

        @pl.when(i != 0)
        def _():
            pooled_ref[...] += row

    return body


def _make_mlp(kh, ka):
    bspec32 = pl.BlockSpec((BN, 32), lambda i: (i, 0))
    bspec16 = pl.BlockSpec((BN, 16), lambda i: (i, 0))
    wspec = pl.BlockSpec((HID, HID), lambda i: (0, 0))
    bvecspec = pl.BlockSpec((1, HID), lambda i: (0, 0))
    return pl.pallas_call(
        _mlp_body_factory(kh, ka),
        grid=(NB,),
        in_specs=[bspec32] * kh + [bspec16] * ka
        + [wspec, bvecspec, wspec, bvecspec],
        out_specs=[bspec32] * 4 + [pl.BlockSpec((8, OUT), lambda i: (0, 0))],
        out_shape=[jax.ShapeDtypeStruct((N, 32), jnp.float32)] * 4
        + [jax.ShapeDtypeStruct((8, OUT), jnp.float32)],
    )


_mlp1 = _make_mlp(3, 5)
_mlp = _make_mlp(4, 8)

_h0_call = pl.pallas_call(
    _h0_body,
    grid=(NB,),
    in_specs=[
        pl.BlockSpec((1, 1, BN), lambda i: (i, 0, 0)),
        pl.BlockSpec((BN, 16), lambda i: (i, 0)),
        pl.BlockSpec((BN, 16), lambda i: (i + NB, 0)),
        pl.BlockSpec((BN, 32), lambda i: (i, 0)),
        pl.BlockSpec((128, 32), lambda i: (0, 0)),
        pl.BlockSpec((520, 16), lambda i: (0, 0)),
    ],
    out_specs=[
        pl.BlockSpec((BN, 32), lambda i: (i, 0)),
        pl.BlockSpec((BN, 32), lambda i: (i, 0)),
        pl.BlockSpec((8, OUT), lambda i: (0, 0)),
    ],
    out_shape=[
        jax.ShapeDtypeStruct((N, 32), jnp.float32),
        jax.ShapeDtypeStruct((N, 32), jnp.float32),
        jax.ShapeDtypeStruct((8, OUT), jnp.float32),
    ],
)


def _readout_body(pooled_ref, w_ref, b_ref, out_ref):
    acc = jnp.zeros((8, OUT), jnp.float32)
    for i in range(NLAYERS + 1):
        acc = acc + lax.dot_general(
            pooled_ref[i], w_ref[i], (((1,), (0,)), ((), ())),
            preferred_element_type=jnp.float32)
    bsum = jnp.sum(b_ref[...], axis=0)
    out_ref[...] = acc + jnp.broadcast_to(bsum[None, :], (8, OUT))


_readout = pl.pallas_call(
    _readout_body,
    out_shape=jax.ShapeDtypeStruct((8, OUT), jnp.float32),
)


def kernel(node_type, pos_undirected, edge_index, ast_table, deg_table,
           gin_w1, gin_b1, gin_w2, gin_b2, pred_w, pred_b):
    src = edge_index[0]
    dst = edge_index[1]
    npad = EP - E
    srcA2d = jnp.concatenate(
        [2 * src, jnp.zeros((npad,), jnp.int32)]).reshape(GROUPS, 128)
    srcB2d = jnp.concatenate(
        [2 * src + 1, jnp.ones((npad,), jnp.int32)]).reshape(GROUPS, 128)
    dst2d = jnp.concatenate(
        [dst, jnp.full((npad,), DUMMY_DST, jnp.int32)]).reshape(GROUPS, 128)

    zeros16 = jnp.zeros((128, 16), jnp.float32)
    ones16 = jnp.ones((128, 16), jnp.float32)

    deg2 = _deg_kernel(dst2d, ones16, zeros16)

    nt3 = node_type.reshape(NB, 1, BN)
    ast_pad = jnp.zeros((128, 32), jnp.float32).at[:101].set(ast_table)
    degtab_pad = jnp.zeros((520, 16), jnp.float32).at[:513].set(deg_table)
    s1, s2, pooled0 = _h0_call(nt3, deg2, deg2, pos_undirected,
                               ast_pad, degtab_pad)

    pooleds = [pooled0]
    h0v = [pos_undirected.reshape(2 * N, 16), s1.reshape(2 * N, 16),
           s2.reshape(2 * N, 16)]
    a = _segsum5(h0v[0], h0v[1], h0v[2], srcA2d, srcB2d, dst2d, zeros16)
    w1p = jnp.zeros((HID, HID), jnp.float32).at[:80].set(gin_w1[0])
    z0, z1, z2, z3, pooled = _mlp1(
        pos_undirected, s1, s2, a[0], a[1], a[2], a[3], a[4],
        w1p, gin_b1[0].reshape(1, HID), gin_w2[0], gin_b2[0].reshape(1, HID))
    pooleds.append(pooled)
    hs = [z0, z1, z2, z3]
    for li in range(1, NLAYERS):
        hv = [h.reshape(2 * N, 16) for h in hs]
        a = _segsum8(hv[0], hv[1], hv[2], hv[3],
                     srcA2d, srcB2d, dst2d, zeros16)
        z0, z1, z2, z3, pooled = _mlp(
            hs[0], hs[1], hs[2], hs[3],
            a[0], a[1], a[2], a[3], a[4], a[5], a[6], a[7],
            gin_w1[li], gin_b1[li].reshape(1, HID),
            gin_w2[li], gin_b2[li].reshape(1, HID))
        pooleds.append(pooled)
        hs = [z0, z1, z2, z3]

    pooled_arr = jnp.stack(pooleds)
    w_pad = [jnp.zeros((OUT, OUT), jnp.float32).at[:80].set(pred_w[0])]
    w_pad += [pred_w[i] for i in range(1, NLAYERS + 1)]
    w_arr = jnp.stack(w_pad)
    b_arr = jnp.zeros((8, OUT), jnp.float32).at[:NLAYERS + 1].set(
        jnp.stack(pred_b))
    out = _readout(pooled_arr, w_arr, b_arr)
    return out[:1]
